# Initial kernel scaffold; baseline (speedup 1.0000x reference)
import jax
import jax.numpy as jnp
from jax.experimental import pallas as pl


def kernel(weights, z_vals, ray_indices, num_rays):
    raise NotImplementedError("write your pallas kernel here")


# speedup vs baseline: 24.4027x; 24.4027x over previous
import functools

import jax
import jax.numpy as jnp
from jax import lax
from jax.experimental import pallas as pl
from jax.experimental.pallas import tpu as pltpu
from jax.experimental.pallas import tpu_sc as plsc

N = 6_400_000
R = 100_000
LANES = 16
ROW = 128
ROWS_PER_BLK = 16
BLK = ROWS_PER_BLK * ROW
NBLK = N // BLK
NC, NS = 2, 16
NW = NC * NS
ACC = 100_352
SLICE = ACC // NS
FIN_ROWS = ACC // 128


def _sc_body(w_hbm, z_hbm, idx_hbm, out_num, out_den, out_min, out_max,
             acc_num, acc_den, idx_buf, w_buf, z_buf, wz_buf, zero_buf,
             vmin_ref, vmax_ref, sem):
  c = lax.axis_index("c")
  s = lax.axis_index("s")
  wid = s * NC + c

  @pl.loop(0, SLICE // LANES, unroll=8)
  def _zero(i):
    zero_buf[pl.ds(i * LANES, LANES)] = jnp.zeros((LANES,), jnp.float32)

  pltpu.sync_copy(zero_buf, acc_num.at[pl.ds(s * SLICE, SLICE)])
  pltpu.sync_copy(zero_buf, acc_den.at[pl.ds(s * SLICE, SLICE)])
  vmin_ref[...] = jnp.full((LANES,), jnp.inf, jnp.float32)
  vmax_ref[...] = jnp.full((LANES,), -jnp.inf, jnp.float32)
  plsc.subcore_barrier()

  nblk_w = jnp.where(wid < NBLK % NW, NBLK // NW + 1, NBLK // NW)

  @pl.loop(0, nblk_w)
  def _blk(t):
    b = t * NW + wid
    pltpu.sync_copy(idx_hbm.at[b], idx_buf)
    pltpu.sync_copy(w_hbm.at[b], w_buf)
    pltpu.sync_copy(z_hbm.at[b], z_buf)

    vmin = vmin_ref[...]
    vmax = vmax_ref[...]
    for j in range(ROWS_PER_BLK):
      for v in range(ROW // LANES):
        sl = pl.ds(v * LANES, LANES)
        wv = w_buf[j, sl]
        zv = z_buf[j, sl]
        wz_buf[j, sl] = wv * zv
        vmin = jnp.minimum(vmin, zv)
        vmax = jnp.maximum(vmax, zv)
    vmin_ref[...] = vmin
    vmax_ref[...] = vmax

    descs = []
    for j in range(ROWS_PER_BLK):
      idx_row = idx_buf.at[j]
      descs.append(
          pltpu.async_copy(wz_buf.at[j], acc_num.at[idx_row], sem, add=True))
      descs.append(
          pltpu.async_copy(w_buf.at[j], acc_den.at[idx_row], sem, add=True))
    for d in descs:
      d.wait()

  plsc.subcore_barrier()

  sl = pl.ds(s * SLICE, SLICE)
  pltpu.sync_copy(acc_num.at[sl], out_num.at[c, sl])
  pltpu.sync_copy(acc_den.at[sl], out_den.at[c, sl])
  pltpu.sync_copy(vmin_ref, out_min.at[wid])
  pltpu.sync_copy(vmax_ref, out_max.at[wid])


@jax.jit
def _sc_segment_sums(w3, z3, idx3):
  mesh = plsc.VectorSubcoreMesh(core_axis_name="c", subcore_axis_name="s")
  f = pl.kernel(
      _sc_body,
      out_type=(
          jax.ShapeDtypeStruct((NC, ACC), jnp.float32),
          jax.ShapeDtypeStruct((NC, ACC), jnp.float32),
          jax.ShapeDtypeStruct((NW, LANES), jnp.float32),
          jax.ShapeDtypeStruct((NW, LANES), jnp.float32),
      ),
      mesh=mesh,
      scratch_types=(
          pltpu.VMEM_SHARED((ACC,), jnp.float32),
          pltpu.VMEM_SHARED((ACC,), jnp.float32),
          pltpu.VMEM((ROWS_PER_BLK, ROW), jnp.int32),
          pltpu.VMEM((ROWS_PER_BLK, ROW), jnp.float32),
          pltpu.VMEM((ROWS_PER_BLK, ROW), jnp.float32),
          pltpu.VMEM((ROWS_PER_BLK, ROW), jnp.float32),
          pltpu.VMEM((SLICE,), jnp.float32),
          pltpu.VMEM((LANES,), jnp.float32),
          pltpu.VMEM((LANES,), jnp.float32),
          pltpu.SemaphoreType.DMA,
      ),
  )
  return f(w3, z3, idx3)


def _fin_body(num_ref, den_ref, mn_ref, mx_ref, out_ref):
  zmin = jnp.min(mn_ref[...])
  zmax = jnp.max(mx_ref[...])
  num = num_ref[0] + num_ref[1]
  den = den_ref[0] + den_ref[1]
  out_ref[...] = jnp.clip(num / (den + 1e-10), zmin, zmax)


@jax.jit
def _finalize(num_p, den_p, mn, mx):
  return pl.pallas_call(
      _fin_body,
      out_shape=jax.ShapeDtypeStruct((FIN_ROWS, 128), jnp.float32),
  )(num_p, den_p, mn, mx)


def kernel(weights, z_vals, ray_indices, num_rays):
  del num_rays
  w3 = weights.reshape(NBLK, ROWS_PER_BLK, ROW)
  z3 = z_vals.reshape(NBLK, ROWS_PER_BLK, ROW)
  idx3 = ray_indices.astype(jnp.int32).reshape(NBLK, ROWS_PER_BLK, ROW)
  out_num, out_den, out_min, out_max = _sc_segment_sums(w3, z3, idx3)
  depth = _finalize(
      out_num.reshape(NC, FIN_ROWS, 128),
      out_den.reshape(NC, FIN_ROWS, 128),
      out_min, out_max)
  return depth.reshape(ACC)[:R].reshape(R, 1)


# speedup vs baseline: 43.3444x; 1.7762x over previous
import jax
import jax.numpy as jnp
from jax import lax
from jax.experimental import pallas as pl
from jax.experimental.pallas import tpu as pltpu
from jax.experimental.pallas import tpu_sc as plsc

N = 6_400_000
R = 100_000
LANES = 16
ROW = 128
ROWS_PER_BLK = 16
BLK = ROWS_PER_BLK * ROW
NBLK = N // BLK
NC, NS = 2, 16
NW = NC * NS
ACC = 100_352
SLICE = ACC // NS
FIN_ROWS = ACC // 128
NBUF = 3
LOOP_ITERS = 99


def _sc_body(w_hbm, z_hbm, idx_hbm, out_num, out_den, out_min, out_max,
             acc_num, acc_den, idx_buf, w_buf, z_buf, wz_buf, zero_buf,
             vmin_ref, vmax_ref, drain_buf, in_sems, scat_sems):
  c = lax.axis_index("c")
  s = lax.axis_index("s")
  wid = s * NC + c

  @pl.loop(0, SLICE // LANES, unroll=8)
  def _zero(i):
    zero_buf[pl.ds(i * LANES, LANES)] = jnp.zeros((LANES,), jnp.float32)

  pltpu.sync_copy(zero_buf, acc_num.at[pl.ds(s * SLICE, SLICE)])
  pltpu.sync_copy(zero_buf, acc_den.at[pl.ds(s * SLICE, SLICE)])
  vmin_ref[...] = jnp.full((LANES,), jnp.inf, jnp.float32)
  vmax_ref[...] = jnp.full((LANES,), -jnp.inf, jnp.float32)
  plsc.subcore_barrier()

  nblk_w = jnp.where(wid < NBLK % NW, NBLK // NW + 1, NBLK // NW)

  def fire_in(k, slot):
    b = k * NW + wid
    pltpu.async_copy(idx_hbm.at[b], idx_buf.at[slot], in_sems.at[slot])
    pltpu.async_copy(w_hbm.at[b], w_buf.at[slot], in_sems.at[slot])
    pltpu.async_copy(z_hbm.at[b], z_buf.at[slot], in_sems.at[slot])

  def wait_in(k, slot):
    b = k * NW + wid
    pltpu.make_async_copy(idx_hbm.at[b], idx_buf.at[slot],
                          in_sems.at[slot]).wait()
    pltpu.make_async_copy(w_hbm.at[b], w_buf.at[slot],
                          in_sems.at[slot]).wait()
    pltpu.make_async_copy(z_hbm.at[b], z_buf.at[slot],
                          in_sems.at[slot]).wait()

  def drain_scatters(slot):
    pltpu.make_async_copy(w_hbm.at[pl.ds(0, 2)], drain_buf,
                          scat_sems.at[slot]).wait()

  def process(k, slot):
    wait_in(k, slot)
    vmin = vmin_ref[...]
    vmax = vmax_ref[...]
    for j in range(ROWS_PER_BLK):
      for v in range(ROW // LANES):
        sl = pl.ds(v * LANES, LANES)
        wv = w_buf[slot, j, sl]
        zv = z_buf[slot, j, sl]
        wz_buf[slot, j, sl] = wv * zv
        vmin = jnp.minimum(vmin, zv)
        vmax = jnp.maximum(vmax, zv)
    vmin_ref[...] = vmin
    vmax_ref[...] = vmax
    for j in range(ROWS_PER_BLK):
      idx_row = idx_buf.at[slot, j]
      pltpu.async_copy(wz_buf.at[slot, j], acc_num.at[idx_row],
                       scat_sems.at[slot], add=True)
      pltpu.async_copy(w_buf.at[slot, j], acc_den.at[idx_row],
                       scat_sems.at[slot], add=True)

  @pl.when(0 < nblk_w)
  def _prime():
    fire_in(0, 0)

  @pl.loop(0, LOOP_ITERS, step=NBUF)
  def _blk(t):
    for b in range(NBUF):
      k = t + b
      slot = b
      nxt = (b + 1) % NBUF

      @pl.when((k >= 2) & (k - 2 < nblk_w - 2))
      def _():
        drain_scatters(nxt)

      @pl.when(k + 1 < nblk_w)
      def _():
        fire_in(k + 1, nxt)

      @pl.when(k < nblk_w)
      def _():
        process(k, slot)

  @pl.loop(0, 2)
  def _tail(i):
    j = nblk_w - 2 + i
    for slot in range(NBUF):
      @pl.when(j % NBUF == slot)
      def _():
        drain_scatters(slot)

  plsc.subcore_barrier()

  sl = pl.ds(s * SLICE, SLICE)
  pltpu.sync_copy(acc_num.at[sl], out_num.at[c, sl])
  pltpu.sync_copy(acc_den.at[sl], out_den.at[c, sl])
  pltpu.sync_copy(vmin_ref, out_min.at[wid])
  pltpu.sync_copy(vmax_ref, out_max.at[wid])


@jax.jit
def _sc_segment_sums(w3, z3, idx3):
  mesh = plsc.VectorSubcoreMesh(core_axis_name="c", subcore_axis_name="s")
  f = pl.kernel(
      _sc_body,
      out_type=(
          jax.ShapeDtypeStruct((NC, ACC), jnp.float32),
          jax.ShapeDtypeStruct((NC, ACC), jnp.float32),
          jax.ShapeDtypeStruct((NW, LANES), jnp.float32),
          jax.ShapeDtypeStruct((NW, LANES), jnp.float32),
      ),
      mesh=mesh,
      scratch_types=(
          pltpu.VMEM_SHARED((ACC,), jnp.float32),
          pltpu.VMEM_SHARED((ACC,), jnp.float32),
          pltpu.VMEM((NBUF, ROWS_PER_BLK, ROW), jnp.int32),
          pltpu.VMEM((NBUF, ROWS_PER_BLK, ROW), jnp.float32),
          pltpu.VMEM((NBUF, ROWS_PER_BLK, ROW), jnp.float32),
          pltpu.VMEM((NBUF, ROWS_PER_BLK, ROW), jnp.float32),
          pltpu.VMEM((SLICE,), jnp.float32),
          pltpu.VMEM((LANES,), jnp.float32),
          pltpu.VMEM((LANES,), jnp.float32),
          pltpu.VMEM((2, ROWS_PER_BLK, ROW), jnp.float32),
          pltpu.SemaphoreType.DMA((NBUF,)),
          pltpu.SemaphoreType.DMA((NBUF,)),
      ),
  )
  return f(w3, z3, idx3)


def _fin_body(num_ref, den_ref, mn_ref, mx_ref, out_ref):
  zmin = jnp.min(mn_ref[...])
  zmax = jnp.max(mx_ref[...])
  num = num_ref[0] + num_ref[1]
  den = den_ref[0] + den_ref[1]
  out_ref[...] = jnp.clip(num / (den + 1e-10), zmin, zmax)


@jax.jit
def _finalize(num_p, den_p, mn, mx):
  return pl.pallas_call(
      _fin_body,
      out_shape=jax.ShapeDtypeStruct((FIN_ROWS, 128), jnp.float32),
  )(num_p, den_p, mn, mx)


def kernel(weights, z_vals, ray_indices, num_rays):
  del num_rays
  w3 = weights.reshape(NBLK, ROWS_PER_BLK, ROW)
  z3 = z_vals.reshape(NBLK, ROWS_PER_BLK, ROW)
  idx3 = ray_indices.astype(jnp.int32).reshape(NBLK, ROWS_PER_BLK, ROW)
  out_num, out_den, out_min, out_max = _sc_segment_sums(w3, z3, idx3)
  depth = _finalize(
      out_num.reshape(NC, FIN_ROWS, 128),
      out_den.reshape(NC, FIN_ROWS, 128),
      out_min, out_max)
  return depth.reshape(ACC)[:R].reshape(R, 1)


# speedup vs baseline: 45.2211x; 1.0433x over previous
import jax
import jax.numpy as jnp
from jax import lax
from jax.experimental import pallas as pl
from jax.experimental.pallas import tpu as pltpu
from jax.experimental.pallas import tpu_sc as plsc

N = 6_400_000
R = 100_000
LANES = 16
ROW = 128
BLK = 2048
VECS = BLK // LANES
NBLK = N // BLK
NC, NS = 2, 16
NW = NC * NS
ACC = 100_352
SLICE = ACC // NS
FIN_ROWS = ACC // 128
NBUF = 3
LOOP_ITERS = 99
CAP = 4224


def _sc_body(w_hbm, z_hbm, idx_hbm, out_num, out_den, out_min, out_max,
             acc_num, acc_den,
             idx_b0, idx_b1, idx_b2, w_b0, w_b1, w_b2, z_b0, z_b1, z_b2,
             si_b0, si_b1, si_b2, s1_b0, s1_b1, s1_b2, s2_b0, s2_b1, s2_b2,
             zero_buf, vmin_ref, vmax_ref, drain_buf, rows_ref, in_sems,
             scat_sems):
  idx_bufs = (idx_b0, idx_b1, idx_b2)
  w_bufs = (w_b0, w_b1, w_b2)
  z_bufs = (z_b0, z_b1, z_b2)
  sidxs = (si_b0, si_b1, si_b2)
  sv1s = (s1_b0, s1_b1, s1_b2)
  sv2s = (s2_b0, s2_b1, s2_b2)
  c = lax.axis_index("c")
  s = lax.axis_index("s")
  wid = s * NC + c
  iota = lax.iota(jnp.int32, LANES)
  is15 = iota == (LANES - 1)
  not15 = jnp.logical_not(is15)

  @pl.loop(0, SLICE // LANES, unroll=8)
  def _zero(i):
    zero_buf[pl.ds(i * LANES, LANES)] = jnp.zeros((LANES,), jnp.float32)

  pltpu.sync_copy(zero_buf, acc_num.at[pl.ds(s * SLICE, SLICE)])
  pltpu.sync_copy(zero_buf, acc_den.at[pl.ds(s * SLICE, SLICE)])
  vmin_ref[...] = jnp.full((LANES,), jnp.inf, jnp.float32)
  vmax_ref[...] = jnp.full((LANES,), -jnp.inf, jnp.float32)
  plsc.subcore_barrier()

  nblk_w = jnp.where(wid < NBLK % NW, NBLK // NW + 1, NBLK // NW)

  def fire_in(k, slot):
    b = (k * NW + wid) * BLK
    pltpu.async_copy(idx_hbm.at[pl.ds(b, BLK)],
                     idx_bufs[slot].at[pl.ds(0, BLK)], in_sems.at[slot])
    pltpu.async_copy(w_hbm.at[pl.ds(b, BLK)], w_bufs[slot],
                     in_sems.at[slot])
    pltpu.async_copy(z_hbm.at[pl.ds(b, BLK)], z_bufs[slot],
                     in_sems.at[slot])

  def wait_in(k, slot):
    b = (k * NW + wid) * BLK
    pltpu.make_async_copy(idx_hbm.at[pl.ds(b, BLK)],
                          idx_bufs[slot].at[pl.ds(0, BLK)],
                          in_sems.at[slot]).wait()
    pltpu.make_async_copy(w_hbm.at[pl.ds(b, BLK)], w_bufs[slot],
                          in_sems.at[slot]).wait()
    pltpu.make_async_copy(z_hbm.at[pl.ds(b, BLK)], z_bufs[slot],
                          in_sems.at[slot]).wait()

  def drain_scatters(slot):
    nr = rows_ref[slot]

    @pl.loop(0, nr)
    def _(r):
      pltpu.make_async_copy(w_hbm.at[pl.ds(0, 2 * ROW)], drain_buf,
                            scat_sems.at[slot]).wait()

  def process(k, slot):
    wait_in(k, slot)
    zeros = jnp.zeros((LANES,), jnp.float32)

    @pl.loop(0, VECS, init_carry=(jnp.int32(0), vmin_ref[...], vmax_ref[...]),
             unroll=4)
    def _vec(v, carry):
      p, vmin, vmax = carry
      base = v * LANES
      idx = idx_bufs[slot][pl.ds(base, LANES)]
      idxn = idx_bufs[slot][pl.ds(base + 1, LANES)]
      wv = w_bufs[slot][pl.ds(base, LANES)]
      zv = z_bufs[slot][pl.ds(base, LANES)]
      wz = wv * zv
      vmin = jnp.minimum(vmin, zv)
      vmax = jnp.maximum(vmax, zv)
      c1 = plsc.cumsum(wv)
      c2 = plsc.cumsum(wz)
      mneq = idx != idxn
      m = jnp.logical_or(mneq, is15)
      m2 = jnp.logical_and(mneq, not15)
      plsc.store_compressed(sidxs[slot].at[pl.ds(p, LANES)], idx, mask=m)
      plsc.store_compressed(sv1s[slot].at[pl.ds(p, LANES)], c1, mask=m)
      plsc.store_compressed(sv2s[slot].at[pl.ds(p, LANES)], c2, mask=m)
      q = p + jnp.sum(m.astype(jnp.int32))
      plsc.store_compressed(sidxs[slot].at[pl.ds(q, LANES)], idxn, mask=m2)
      plsc.store_compressed(sv1s[slot].at[pl.ds(q, LANES)], -c1, mask=m2)
      plsc.store_compressed(sv2s[slot].at[pl.ds(q, LANES)], -c2, mask=m2)
      q = q + jnp.sum(m2.astype(jnp.int32))
      return q, vmin, vmax

    p, vmin, vmax = _vec
    vmin_ref[...] = vmin
    vmax_ref[...] = vmax

    rows = lax.shift_right_logical(p + (ROW - 1), 7)
    end = lax.shift_left(rows, 7)
    for t in range(ROW // LANES):
      st = p + t * LANES

      @pl.when(st < end)
      def _():
        sidxs[slot][pl.ds(st, LANES)] = iota + (R + t * LANES)
        sv1s[slot][pl.ds(st, LANES)] = zeros
        sv2s[slot][pl.ds(st, LANES)] = zeros

    @pl.loop(0, rows)
    def _flush(r):
      isl = sidxs[slot].at[pl.ds(r * ROW, ROW)]
      pltpu.async_copy(sv2s[slot].at[pl.ds(r * ROW, ROW)], acc_num.at[isl],
                       scat_sems.at[slot], add=True)
      pltpu.async_copy(sv1s[slot].at[pl.ds(r * ROW, ROW)], acc_den.at[isl],
                       scat_sems.at[slot], add=True)

    rows_ref[slot] = rows

  @pl.when(0 < nblk_w)
  def _prime():
    fire_in(0, 0)

  @pl.loop(0, LOOP_ITERS, step=NBUF)
  def _blk(t):
    for b in range(NBUF):
      k = t + b
      slot = b
      nxt = (b + 1) % NBUF

      @pl.when((k >= 2) & (k - 2 < nblk_w - 2))
      def _():
        drain_scatters(nxt)

      @pl.when(k + 1 < nblk_w)
      def _():
        fire_in(k + 1, nxt)

      @pl.when(k < nblk_w)
      def _():
        process(k, slot)

  @pl.loop(0, 2)
  def _tail(i):
    j = nblk_w - 2 + i
    for slot in range(NBUF):
      @pl.when(j % NBUF == slot)
      def _():
        drain_scatters(slot)

  plsc.subcore_barrier()

  sl = pl.ds(s * SLICE, SLICE)
  pltpu.sync_copy(acc_num.at[sl], out_num.at[c, sl])
  pltpu.sync_copy(acc_den.at[sl], out_den.at[c, sl])
  pltpu.sync_copy(vmin_ref, out_min.at[wid])
  pltpu.sync_copy(vmax_ref, out_max.at[wid])


@jax.jit
def _sc_segment_sums(w1, z1, idx1):
  mesh = plsc.VectorSubcoreMesh(core_axis_name="c", subcore_axis_name="s")
  f = pl.kernel(
      _sc_body,
      out_type=(
          jax.ShapeDtypeStruct((NC, ACC), jnp.float32),
          jax.ShapeDtypeStruct((NC, ACC), jnp.float32),
          jax.ShapeDtypeStruct((NW, LANES), jnp.float32),
          jax.ShapeDtypeStruct((NW, LANES), jnp.float32),
      ),
      mesh=mesh,
      compiler_params=pltpu.CompilerParams(needs_layout_passes=False),
      scratch_types=(
          pltpu.VMEM_SHARED((ACC,), jnp.float32),
          pltpu.VMEM_SHARED((ACC,), jnp.float32),
          pltpu.VMEM((BLK + LANES,), jnp.int32),
          pltpu.VMEM((BLK + LANES,), jnp.int32),
          pltpu.VMEM((BLK + LANES,), jnp.int32),
          pltpu.VMEM((BLK,), jnp.float32),
          pltpu.VMEM((BLK,), jnp.float32),
          pltpu.VMEM((BLK,), jnp.float32),
          pltpu.VMEM((BLK,), jnp.float32),
          pltpu.VMEM((BLK,), jnp.float32),
          pltpu.VMEM((BLK,), jnp.float32),
          pltpu.VMEM((CAP,), jnp.int32),
          pltpu.VMEM((CAP,), jnp.int32),
          pltpu.VMEM((CAP,), jnp.int32),
          pltpu.VMEM((CAP,), jnp.float32),
          pltpu.VMEM((CAP,), jnp.float32),
          pltpu.VMEM((CAP,), jnp.float32),
          pltpu.VMEM((CAP,), jnp.float32),
          pltpu.VMEM((CAP,), jnp.float32),
          pltpu.VMEM((CAP,), jnp.float32),
          pltpu.VMEM((SLICE,), jnp.float32),
          pltpu.VMEM((LANES,), jnp.float32),
          pltpu.VMEM((LANES,), jnp.float32),
          pltpu.VMEM((2 * ROW,), jnp.float32),
          pltpu.SMEM((NBUF,), jnp.int32),
          pltpu.SemaphoreType.DMA((NBUF,)),
          pltpu.SemaphoreType.DMA((NBUF,)),
      ),
  )
  return f(w1, z1, idx1)


def _fin_body(num_ref, den_ref, mn_ref, mx_ref, out_ref):
  zmin = jnp.min(mn_ref[...])
  zmax = jnp.max(mx_ref[...])
  num = num_ref[0] + num_ref[1]
  den = den_ref[0] + den_ref[1]
  out_ref[...] = jnp.clip(num / (den + 1e-10), zmin, zmax)


@jax.jit
def _finalize(num_p, den_p, mn, mx):
  return pl.pallas_call(
      _fin_body,
      out_shape=jax.ShapeDtypeStruct((FIN_ROWS, 128), jnp.float32),
  )(num_p, den_p, mn, mx)


def kernel(weights, z_vals, ray_indices, num_rays):
  del num_rays
  w1 = weights.reshape(N)
  z1 = z_vals.reshape(N)
  idx1 = ray_indices.astype(jnp.int32).reshape(N)
  out_num, out_den, out_min, out_max = _sc_segment_sums(w1, z1, idx1)
  depth = _finalize(
      out_num.reshape(NC, FIN_ROWS, 128),
      out_den.reshape(NC, FIN_ROWS, 128),
      out_min, out_max)
  return depth.reshape(ACC)[:R].reshape(R, 1)


# speedup vs baseline: 50.6990x; 1.1211x over previous
import jax
import jax.numpy as jnp
from jax import lax
from jax.experimental import pallas as pl
from jax.experimental.pallas import tpu as pltpu
from jax.experimental.pallas import tpu_sc as plsc

N = 6_400_000
R = 100_000
LANES = 16
ROW = 128
BLK = 2048
VECS = BLK // LANES
NBLK = N // BLK
NC, NS = 2, 16
NW = NC * NS
ACC = 100_352
SLICE = ACC // NS
FIN_ROWS = ACC // 128
NBUF = 3
LOOP_ITERS = 99
CAP = 4224


def _sc_body(w_hbm, z_hbm, idx_hbm, out_num, out_den, out_min, out_max,
             acc_num, acc_den,
             idx_b0, idx_b1, idx_b2, w_b0, w_b1, w_b2, z_b0, z_b1, z_b2,
             si_b0, si_b1, si_b2, s1_b0, s1_b1, s1_b2, s2_b0, s2_b1, s2_b2,
             zero_buf, vmin_ref, vmax_ref, drain_buf, rows_ref, in_sems,
             scat_sems):
  idx_bufs = (idx_b0, idx_b1, idx_b2)
  w_bufs = (w_b0, w_b1, w_b2)
  z_bufs = (z_b0, z_b1, z_b2)
  sidxs = (si_b0, si_b1, si_b2)
  sv1s = (s1_b0, s1_b1, s1_b2)
  sv2s = (s2_b0, s2_b1, s2_b2)
  c = lax.axis_index("c")
  s = lax.axis_index("s")
  wid = s * NC + c
  iota = lax.iota(jnp.int32, LANES)
  is15 = iota == (LANES - 1)
  not15 = jnp.logical_not(is15)

  @pl.loop(0, SLICE // LANES, unroll=8)
  def _zero(i):
    zero_buf[pl.ds(i * LANES, LANES)] = jnp.zeros((LANES,), jnp.float32)

  pltpu.sync_copy(zero_buf, acc_num.at[pl.ds(s * SLICE, SLICE)])
  pltpu.sync_copy(zero_buf, acc_den.at[pl.ds(s * SLICE, SLICE)])
  vmin_ref[...] = jnp.full((LANES,), jnp.inf, jnp.float32)
  vmax_ref[...] = jnp.full((LANES,), -jnp.inf, jnp.float32)
  plsc.subcore_barrier()

  nblk_w = jnp.where(wid < NBLK % NW, NBLK // NW + 1, NBLK // NW)

  def fire_in(k, slot):
    b = (k * NW + wid) * BLK
    pltpu.async_copy(idx_hbm.at[pl.ds(b, BLK)],
                     idx_bufs[slot].at[pl.ds(0, BLK)], in_sems.at[slot])
    pltpu.async_copy(w_hbm.at[pl.ds(b, BLK)], w_bufs[slot],
                     in_sems.at[slot])
    pltpu.async_copy(z_hbm.at[pl.ds(b, BLK)], z_bufs[slot],
                     in_sems.at[slot])

  def wait_in(k, slot):
    b = (k * NW + wid) * BLK
    pltpu.make_async_copy(idx_hbm.at[pl.ds(b, BLK)],
                          idx_bufs[slot].at[pl.ds(0, BLK)],
                          in_sems.at[slot]).wait()
    pltpu.make_async_copy(w_hbm.at[pl.ds(b, BLK)], w_bufs[slot],
                          in_sems.at[slot]).wait()
    pltpu.make_async_copy(z_hbm.at[pl.ds(b, BLK)], z_bufs[slot],
                          in_sems.at[slot]).wait()

  def drain_scatters(slot):
    nr = rows_ref[slot]

    @pl.loop(0, nr)
    def _(r):
      pltpu.make_async_copy(w_hbm.at[pl.ds(0, 2 * ROW)], drain_buf,
                            scat_sems.at[slot]).wait()

  def process(k, slot):
    wait_in(k, slot)
    zeros = jnp.zeros((LANES,), jnp.float32)

    @pl.loop(0, VECS, init_carry=(jnp.int32(0), vmin_ref[...], vmax_ref[...]),
             unroll=8)
    def _vec(v, carry):
      p, vmin, vmax = carry
      base = v * LANES
      idx = idx_bufs[slot][pl.ds(base, LANES)]
      idxn = idx_bufs[slot][pl.ds(base + 1, LANES)]
      wv = w_bufs[slot][pl.ds(base, LANES)]
      zv = z_bufs[slot][pl.ds(base, LANES)]
      wz = wv * zv
      vmin = jnp.minimum(vmin, zv)
      vmax = jnp.maximum(vmax, zv)
      c1 = plsc.cumsum(wv)
      c2 = plsc.cumsum(wz)
      mneq = idx != idxn
      m = jnp.logical_or(mneq, is15)
      m2 = jnp.logical_and(mneq, not15)
      plsc.store_compressed(sidxs[slot].at[pl.ds(p, LANES)], idx, mask=m)
      plsc.store_compressed(sv1s[slot].at[pl.ds(p, LANES)], c1, mask=m)
      plsc.store_compressed(sv2s[slot].at[pl.ds(p, LANES)], c2, mask=m)
      q = p + plsc.all_reduce_population_count(m)[0]
      plsc.store_compressed(sidxs[slot].at[pl.ds(q, LANES)], idxn, mask=m2)
      plsc.store_compressed(sv1s[slot].at[pl.ds(q, LANES)], -c1, mask=m2)
      plsc.store_compressed(sv2s[slot].at[pl.ds(q, LANES)], -c2, mask=m2)
      q = q + plsc.all_reduce_population_count(m2)[0]
      return q, vmin, vmax

    p, vmin, vmax = _vec
    vmin_ref[...] = vmin
    vmax_ref[...] = vmax

    rows = lax.shift_right_logical(p + (ROW - 1), 7)
    end = lax.shift_left(rows, 7)
    for t in range(ROW // LANES):
      st = p + t * LANES

      @pl.when(st < end)
      def _():
        sidxs[slot][pl.ds(st, LANES)] = iota + (R + t * LANES)
        sv1s[slot][pl.ds(st, LANES)] = zeros
        sv2s[slot][pl.ds(st, LANES)] = zeros

    @pl.loop(0, rows)
    def _flush(r):
      isl = sidxs[slot].at[pl.ds(r * ROW, ROW)]
      pltpu.async_copy(sv2s[slot].at[pl.ds(r * ROW, ROW)], acc_num.at[isl],
                       scat_sems.at[slot], add=True)
      pltpu.async_copy(sv1s[slot].at[pl.ds(r * ROW, ROW)], acc_den.at[isl],
                       scat_sems.at[slot], add=True)

    rows_ref[slot] = rows

  @pl.when(0 < nblk_w)
  def _prime():
    fire_in(0, 0)

  @pl.loop(0, LOOP_ITERS, step=NBUF)
  def _blk(t):
    for b in range(NBUF):
      k = t + b
      slot = b
      nxt = (b + 1) % NBUF

      @pl.when((k >= 2) & (k - 2 < nblk_w - 2))
      def _():
        drain_scatters(nxt)

      @pl.when(k + 1 < nblk_w)
      def _():
        fire_in(k + 1, nxt)

      @pl.when(k < nblk_w)
      def _():
        process(k, slot)

  @pl.loop(0, 2)
  def _tail(i):
    j = nblk_w - 2 + i
    for slot in range(NBUF):
      @pl.when(j % NBUF == slot)
      def _():
        drain_scatters(slot)

  plsc.subcore_barrier()

  sl = pl.ds(s * SLICE, SLICE)
  pltpu.sync_copy(acc_num.at[sl], out_num.at[c, sl])
  pltpu.sync_copy(acc_den.at[sl], out_den.at[c, sl])
  pltpu.sync_copy(vmin_ref, out_min.at[wid])
  pltpu.sync_copy(vmax_ref, out_max.at[wid])


@jax.jit
def _sc_segment_sums(w1, z1, idx1):
  mesh = plsc.VectorSubcoreMesh(core_axis_name="c", subcore_axis_name="s")
  f = pl.kernel(
      _sc_body,
      out_type=(
          jax.ShapeDtypeStruct((NC, ACC), jnp.float32),
          jax.ShapeDtypeStruct((NC, ACC), jnp.float32),
          jax.ShapeDtypeStruct((NW, LANES), jnp.float32),
          jax.ShapeDtypeStruct((NW, LANES), jnp.float32),
      ),
      mesh=mesh,
      compiler_params=pltpu.CompilerParams(needs_layout_passes=False),
      scratch_types=(
          pltpu.VMEM_SHARED((ACC,), jnp.float32),
          pltpu.VMEM_SHARED((ACC,), jnp.float32),
          pltpu.VMEM((BLK + LANES,), jnp.int32),
          pltpu.VMEM((BLK + LANES,), jnp.int32),
          pltpu.VMEM((BLK + LANES,), jnp.int32),
          pltpu.VMEM((BLK,), jnp.float32),
          pltpu.VMEM((BLK,), jnp.float32),
          pltpu.VMEM((BLK,), jnp.float32),
          pltpu.VMEM((BLK,), jnp.float32),
          pltpu.VMEM((BLK,), jnp.float32),
          pltpu.VMEM((BLK,), jnp.float32),
          pltpu.VMEM((CAP,), jnp.int32),
          pltpu.VMEM((CAP,), jnp.int32),
          pltpu.VMEM((CAP,), jnp.int32),
          pltpu.VMEM((CAP,), jnp.float32),
          pltpu.VMEM((CAP,), jnp.float32),
          pltpu.VMEM((CAP,), jnp.float32),
          pltpu.VMEM((CAP,), jnp.float32),
          pltpu.VMEM((CAP,), jnp.float32),
          pltpu.VMEM((CAP,), jnp.float32),
          pltpu.VMEM((SLICE,), jnp.float32),
          pltpu.VMEM((LANES,), jnp.float32),
          pltpu.VMEM((LANES,), jnp.float32),
          pltpu.VMEM((2 * ROW,), jnp.float32),
          pltpu.SMEM((NBUF,), jnp.int32),
          pltpu.SemaphoreType.DMA((NBUF,)),
          pltpu.SemaphoreType.DMA((NBUF,)),
      ),
  )
  return f(w1, z1, idx1)


def _fin_body(num_ref, den_ref, mn_ref, mx_ref, out_ref):
  zmin = jnp.min(mn_ref[...])
  zmax = jnp.max(mx_ref[...])
  num = num_ref[0] + num_ref[1]
  den = den_ref[0] + den_ref[1]
  out_ref[...] = jnp.clip(num / (den + 1e-10), zmin, zmax)


@jax.jit
def _finalize(num_p, den_p, mn, mx):
  return pl.pallas_call(
      _fin_body,
      out_shape=jax.ShapeDtypeStruct((FIN_ROWS, 128), jnp.float32),
  )(num_p, den_p, mn, mx)


def kernel(weights, z_vals, ray_indices, num_rays):
  del num_rays
  w1 = weights.reshape(N)
  z1 = z_vals.reshape(N)
  idx1 = ray_indices.astype(jnp.int32).reshape(N)
  out_num, out_den, out_min, out_max = _sc_segment_sums(w1, z1, idx1)
  depth = _finalize(
      out_num.reshape(NC, FIN_ROWS, 128),
      out_den.reshape(NC, FIN_ROWS, 128),
      out_min, out_max)
  return depth.reshape(ACC)[:R].reshape(R, 1)
